# Initial kernel scaffold; baseline (speedup 1.0000x reference)
import jax
import jax.numpy as jnp
from jax.experimental import pallas as pl


def kernel(data, edge_index, W1, b1, W2, b2):
    raise NotImplementedError("write your pallas kernel here")


# speedup vs baseline: 13.5060x; 13.5060x over previous
import functools
import jax
import jax.numpy as jnp
from jax import lax
from jax.experimental import pallas as pl
from jax.experimental.pallas import tpu as pltpu
from jax.experimental.pallas import tpu_sc as plsc

N = 10000
E = 320000
D_IN = 128
HIDDEN = 128
CLASSES = 64

NCORES = 2
NSUB = 16
NW = NCORES * NSUB
E_PER = E // NW
CHUNK = 80
NITER = E_PER // CHUNK
SPT = 632
N_ACC = NSUB * SPT
N_PAD = 10240
DPT = N_PAD // NSUB


def _make_deg_kernel():
    mesh = plsc.VectorSubcoreMesh(core_axis_name="c", subcore_axis_name="s")

    @functools.partial(
        pl.kernel,
        mesh=mesh,
        out_type=jax.ShapeDtypeStruct((NCORES * N_PAD,), jnp.float32),
        scratch_types=[
            pltpu.VMEM_SHARED((N_PAD,), jnp.float32),
            pltpu.VMEM((CHUNK,), jnp.int32),
            pltpu.VMEM((CHUNK,), jnp.float32),
        ],
    )
    def deg_kernel(dst_hbm, zeros_hbm, out_hbm, acc, dst_v, ones_v):
        c = lax.axis_index("c")
        s = lax.axis_index("s")
        pltpu.sync_copy(zeros_hbm, acc.at[pl.ds(s * DPT, DPT)])
        for j in range(CHUNK // 16):
            ones_v[pl.ds(j * 16, 16)] = jnp.full((16,), 1.0, jnp.float32)
        plsc.subcore_barrier()
        base = (c * NSUB + s) * E_PER

        def body(i, carry):
            off = base + i * CHUNK
            pltpu.sync_copy(dst_hbm.at[pl.ds(off, CHUNK)], dst_v)
            pltpu.sync_copy(ones_v, acc.at[dst_v], add=True)
            return carry

        lax.fori_loop(0, NITER, body, 0)
        plsc.subcore_barrier()
        pltpu.sync_copy(
            acc.at[pl.ds(s * DPT, DPT)],
            out_hbm.at[pl.ds(c * N_PAD + s * DPT, DPT)],
        )

    return deg_kernel


def _make_agg_kernel(D):
    mesh = plsc.VectorSubcoreMesh(core_axis_name="c", subcore_axis_name="s")
    extra = {}
    if D != 128:
        extra["compiler_params"] = pltpu.CompilerParams(use_tc_tiling_on_sc=False)

    @functools.partial(
        pl.kernel,
        mesh=mesh,
        **extra,
        out_type=jax.ShapeDtypeStruct((NCORES * N_ACC, D), jnp.float32),
        scratch_types=[
            pltpu.VMEM_SHARED((N_ACC, D), jnp.float32),
            pltpu.VMEM((CHUNK,), jnp.int32),
            pltpu.VMEM((CHUNK,), jnp.int32),
            pltpu.VMEM((CHUNK, D), jnp.float32),
            pltpu.SemaphoreType.DMA,
        ],
    )
    def agg_kernel(xs_hbm, src_hbm, dst_hbm, zeros_hbm, out_hbm,
                   acc, src_v, dst_v, rows_v, sem):
        c = lax.axis_index("c")
        s = lax.axis_index("s")
        pltpu.sync_copy(zeros_hbm, acc.at[pl.ds(s * SPT, SPT)])
        plsc.subcore_barrier()
        base = (c * NSUB + s) * E_PER

        def body(i, carry):
            off = base + i * CHUNK
            pltpu.sync_copy(src_hbm.at[pl.ds(off, CHUNK)], src_v)
            pltpu.sync_copy(dst_hbm.at[pl.ds(off, CHUNK)], dst_v)
            pltpu.async_copy(xs_hbm.at[src_v], rows_v, sem).wait()
            pltpu.sync_copy(rows_v, acc.at[dst_v], add=True)
            return carry

        lax.fori_loop(0, NITER, body, 0)
        plsc.subcore_barrier()
        pltpu.sync_copy(
            acc.at[pl.ds(s * SPT, SPT)],
            out_hbm.at[pl.ds(c * N_ACC + s * SPT, SPT)],
        )

    return agg_kernel


_deg_call = _make_deg_kernel()
_agg128_call = _make_agg_kernel(HIDDEN)
_agg64_call = _make_agg_kernel(CLASSES)

BM = 2000
GRID = N // BM


def _layer1_body(x_ref, w_ref, d0_ref, d1_ref, xs_ref, dinv_ref):
    deg = d0_ref[...] + d1_ref[...] + 1.0
    dinv = lax.rsqrt(deg)
    h = jnp.dot(x_ref[...], w_ref[...], preferred_element_type=jnp.float32)
    xs_ref[...] = h * dinv
    dinv_ref[...] = dinv


def _tc_layer1(x, W1, d0, d1):
    return pl.pallas_call(
        _layer1_body,
        grid=(GRID,),
        in_specs=[
            pl.BlockSpec((BM, D_IN), lambda i: (i, 0)),
            pl.BlockSpec((D_IN, HIDDEN), lambda i: (0, 0)),
            pl.BlockSpec((BM, 1), lambda i: (i, 0)),
            pl.BlockSpec((BM, 1), lambda i: (i, 0)),
        ],
        out_specs=[
            pl.BlockSpec((BM, HIDDEN), lambda i: (i, 0)),
            pl.BlockSpec((BM, 1), lambda i: (i, 0)),
        ],
        out_shape=[
            jax.ShapeDtypeStruct((N, HIDDEN), jnp.float32),
            jax.ShapeDtypeStruct((N, 1), jnp.float32),
        ],
    )(x, W1, d0, d1)


def _layer2_body(a0_ref, a1_ref, xs_ref, dinv_ref, b_ref, w_ref, out_ref):
    dinv = dinv_ref[...]
    z = dinv * (a0_ref[...] + a1_ref[...] + xs_ref[...]) + b_ref[...]
    o = jnp.maximum(z, 0.0)
    h2 = jnp.dot(o, w_ref[...], preferred_element_type=jnp.float32)
    out_ref[...] = h2 * dinv


def _tc_layer2(a0, a1, xs1, dinv, b1, W2):
    return pl.pallas_call(
        _layer2_body,
        grid=(GRID,),
        in_specs=[
            pl.BlockSpec((BM, HIDDEN), lambda i: (i, 0)),
            pl.BlockSpec((BM, HIDDEN), lambda i: (i, 0)),
            pl.BlockSpec((BM, HIDDEN), lambda i: (i, 0)),
            pl.BlockSpec((BM, 1), lambda i: (i, 0)),
            pl.BlockSpec((1, HIDDEN), lambda i: (0, 0)),
            pl.BlockSpec((HIDDEN, CLASSES), lambda i: (0, 0)),
        ],
        out_specs=pl.BlockSpec((BM, CLASSES), lambda i: (i, 0)),
        out_shape=jax.ShapeDtypeStruct((N, CLASSES), jnp.float32),
    )(a0, a1, xs1, dinv, b1, W2)


def _final_body(a0_ref, a1_ref, xs_ref, dinv_ref, b_ref, out_ref):
    z = dinv_ref[...] * (a0_ref[...] + a1_ref[...] + xs_ref[...]) + b_ref[...]
    out_ref[...] = z


def _tc_final(a0, a1, xs2, dinv, b2):
    return pl.pallas_call(
        _final_body,
        grid=(GRID,),
        in_specs=[
            pl.BlockSpec((BM, CLASSES), lambda i: (i, 0)),
            pl.BlockSpec((BM, CLASSES), lambda i: (i, 0)),
            pl.BlockSpec((BM, CLASSES), lambda i: (i, 0)),
            pl.BlockSpec((BM, 1), lambda i: (i, 0)),
            pl.BlockSpec((1, CLASSES), lambda i: (0, 0)),
        ],
        out_specs=pl.BlockSpec((BM, CLASSES), lambda i: (i, 0)),
        out_shape=jax.ShapeDtypeStruct((N, CLASSES), jnp.float32),
    )(a0, a1, xs2, dinv, b2)


def kernel(data, edge_index, W1, b1, W2, b2):
    ei = edge_index.astype(jnp.int32)
    src = ei[0]
    dst = ei[1]

    zeros_deg = jnp.zeros((DPT,), jnp.float32)
    zeros128 = jnp.zeros((SPT, HIDDEN), jnp.float32)
    zeros64 = jnp.zeros((SPT, CLASSES), jnp.float32)

    deg_parts = _deg_call(dst, zeros_deg)
    d0 = deg_parts[0 * N_PAD:0 * N_PAD + N].reshape(N, 1)
    d1 = deg_parts[1 * N_PAD:1 * N_PAD + N].reshape(N, 1)

    xs1, dinv = _tc_layer1(data, W1, d0, d1)

    agg1 = _agg128_call(xs1, src, dst, zeros128)
    xs2 = _tc_layer2(agg1[:N], agg1[N_ACC:N_ACC + N], xs1, dinv,
                     b1.reshape(1, HIDDEN), W2)

    agg2 = _agg64_call(xs2, src, dst, zeros64)
    out = _tc_final(agg2[:N], agg2[N_ACC:N_ACC + N], xs2, dinv,
                    b2.reshape(1, CLASSES))
    return out


# speedup vs baseline: 24.9159x; 1.8448x over previous
import functools
import jax
import jax.numpy as jnp
from jax import lax
from jax.experimental import pallas as pl
from jax.experimental.pallas import tpu as pltpu
from jax.experimental.pallas import tpu_sc as plsc

N = 10000
E = 320000
D_IN = 128
HIDDEN = 128
CLASSES = 64

NCORES = 2
NSUB = 16
NW = NCORES * NSUB
E_PER = E // NW
CHUNK = 80
NITER = E_PER // CHUNK
SPT = 632
N_ACC = NSUB * SPT
N_PAD = 10240
DPT = N_PAD // NSUB


_SC_PARAMS = pltpu.CompilerParams(use_tc_tiling_on_sc=False)


def _make_deg_kernel():
    mesh = plsc.VectorSubcoreMesh(core_axis_name="c", subcore_axis_name="s")

    @functools.partial(
        pl.kernel,
        mesh=mesh,
        compiler_params=_SC_PARAMS,
        out_type=jax.ShapeDtypeStruct((NCORES * N_PAD,), jnp.float32),
        scratch_types=[
            pltpu.VMEM_SHARED((N_PAD,), jnp.float32),
            pltpu.VMEM((NITER, CHUNK), jnp.int32),
            pltpu.VMEM((CHUNK,), jnp.float32),
        ],
    )
    def deg_kernel(dstr_hbm, zeros_hbm, out_hbm, acc, dst2d, ones_v):
        c = lax.axis_index("c")
        s = lax.axis_index("s")
        w = c * NSUB + s
        pltpu.sync_copy(zeros_hbm, acc.at[pl.ds(s * DPT, DPT)])
        pltpu.sync_copy(dstr_hbm.at[pl.ds(w * NITER, NITER)], dst2d)
        for j in range(CHUNK // 16):
            ones_v[pl.ds(j * 16, 16)] = jnp.full((16,), 1.0, jnp.float32)
        plsc.subcore_barrier()

        def body(i, carry):
            pltpu.sync_copy(ones_v, acc.at[dst2d.at[i]], add=True)
            return carry

        lax.fori_loop(0, NITER, body, 0)
        plsc.subcore_barrier()
        pltpu.sync_copy(
            acc.at[pl.ds(s * DPT, DPT)],
            out_hbm.at[pl.ds(c * N_PAD + s * DPT, DPT)],
        )

    return deg_kernel


def _make_agg_kernel(D):
    mesh = plsc.VectorSubcoreMesh(core_axis_name="c", subcore_axis_name="s")
    HALF = (NITER - 1) // 2

    @functools.partial(
        pl.kernel,
        mesh=mesh,
        compiler_params=_SC_PARAMS,
        out_type=jax.ShapeDtypeStruct((NCORES * N_ACC, D), jnp.float32),
        scratch_types=[
            pltpu.VMEM_SHARED((N_ACC, D), jnp.float32),
            pltpu.VMEM((NITER, CHUNK), jnp.int32),
            pltpu.VMEM((NITER, CHUNK), jnp.int32),
            pltpu.VMEM((CHUNK, D), jnp.float32),
            pltpu.VMEM((CHUNK, D), jnp.float32),
            pltpu.SemaphoreType.DMA,
            pltpu.SemaphoreType.DMA,
        ],
    )
    def agg_kernel(xs_hbm, srcr_hbm, dstr_hbm, zeros_hbm, out_hbm,
                   acc, src2d, dst2d, rows0, rows1, sem0, sem1):
        c = lax.axis_index("c")
        s = lax.axis_index("s")
        w = c * NSUB + s
        pltpu.sync_copy(zeros_hbm, acc.at[pl.ds(s * SPT, SPT)])
        pltpu.sync_copy(srcr_hbm.at[pl.ds(w * NITER, NITER)], src2d)
        pltpu.sync_copy(dstr_hbm.at[pl.ds(w * NITER, NITER)], dst2d)
        plsc.subcore_barrier()

        def gather(i, buf, sem):
            return pltpu.make_async_copy(xs_hbm.at[src2d.at[i]], buf, sem)

        gather(0, rows0, sem0).start()

        def body(g, carry):
            i0 = 2 * g
            i1 = i0 + 1
            gather(i0, rows0, sem0).wait()
            gather(i1, rows1, sem1).start()
            pltpu.sync_copy(rows0, acc.at[dst2d.at[i0]], add=True)
            gather(i1, rows1, sem1).wait()
            gather(i0 + 2, rows0, sem0).start()
            pltpu.sync_copy(rows1, acc.at[dst2d.at[i1]], add=True)
            return carry

        lax.fori_loop(0, HALF, body, 0)
        gather(NITER - 1, rows0, sem0).wait()
        pltpu.sync_copy(rows0, acc.at[dst2d.at[NITER - 1]], add=True)
        plsc.subcore_barrier()
        pltpu.sync_copy(
            acc.at[pl.ds(s * SPT, SPT)],
            out_hbm.at[pl.ds(c * N_ACC + s * SPT, SPT)],
        )

    return agg_kernel


_deg_call = _make_deg_kernel()
_agg128_call = _make_agg_kernel(HIDDEN)
_agg64_call = _make_agg_kernel(CLASSES)

BM = 2000
GRID = N // BM


def _layer1_body(x_ref, w_ref, d0_ref, d1_ref, xs_ref, dinv_ref):
    deg = d0_ref[...] + d1_ref[...] + 1.0
    dinv = lax.rsqrt(deg)
    h = jnp.dot(x_ref[...], w_ref[...], preferred_element_type=jnp.float32)
    xs_ref[...] = h * dinv
    dinv_ref[...] = dinv


def _tc_layer1(x, W1, d0, d1):
    return pl.pallas_call(
        _layer1_body,
        grid=(GRID,),
        in_specs=[
            pl.BlockSpec((BM, D_IN), lambda i: (i, 0)),
            pl.BlockSpec((D_IN, HIDDEN), lambda i: (0, 0)),
            pl.BlockSpec((BM, 1), lambda i: (i, 0)),
            pl.BlockSpec((BM, 1), lambda i: (i, 0)),
        ],
        out_specs=[
            pl.BlockSpec((BM, HIDDEN), lambda i: (i, 0)),
            pl.BlockSpec((BM, 1), lambda i: (i, 0)),
        ],
        out_shape=[
            jax.ShapeDtypeStruct((N, HIDDEN), jnp.float32),
            jax.ShapeDtypeStruct((N, 1), jnp.float32),
        ],
    )(x, W1, d0, d1)


def _layer2_body(a0_ref, a1_ref, xs_ref, dinv_ref, b_ref, w_ref, out_ref):
    dinv = dinv_ref[...]
    z = dinv * (a0_ref[...] + a1_ref[...] + xs_ref[...]) + b_ref[...]
    o = jnp.maximum(z, 0.0)
    h2 = jnp.dot(o, w_ref[...], preferred_element_type=jnp.float32)
    out_ref[...] = h2 * dinv


def _tc_layer2(a0, a1, xs1, dinv, b1, W2):
    return pl.pallas_call(
        _layer2_body,
        grid=(GRID,),
        in_specs=[
            pl.BlockSpec((BM, HIDDEN), lambda i: (i, 0)),
            pl.BlockSpec((BM, HIDDEN), lambda i: (i, 0)),
            pl.BlockSpec((BM, HIDDEN), lambda i: (i, 0)),
            pl.BlockSpec((BM, 1), lambda i: (i, 0)),
            pl.BlockSpec((1, HIDDEN), lambda i: (0, 0)),
            pl.BlockSpec((HIDDEN, CLASSES), lambda i: (0, 0)),
        ],
        out_specs=pl.BlockSpec((BM, CLASSES), lambda i: (i, 0)),
        out_shape=jax.ShapeDtypeStruct((N, CLASSES), jnp.float32),
    )(a0, a1, xs1, dinv, b1, W2)


def _final_body(a0_ref, a1_ref, xs_ref, dinv_ref, b_ref, out_ref):
    z = dinv_ref[...] * (a0_ref[...] + a1_ref[...] + xs_ref[...]) + b_ref[...]
    out_ref[...] = z


def _tc_final(a0, a1, xs2, dinv, b2):
    return pl.pallas_call(
        _final_body,
        grid=(GRID,),
        in_specs=[
            pl.BlockSpec((BM, CLASSES), lambda i: (i, 0)),
            pl.BlockSpec((BM, CLASSES), lambda i: (i, 0)),
            pl.BlockSpec((BM, CLASSES), lambda i: (i, 0)),
            pl.BlockSpec((BM, 1), lambda i: (i, 0)),
            pl.BlockSpec((1, CLASSES), lambda i: (0, 0)),
        ],
        out_specs=pl.BlockSpec((BM, CLASSES), lambda i: (i, 0)),
        out_shape=jax.ShapeDtypeStruct((N, CLASSES), jnp.float32),
    )(a0, a1, xs2, dinv, b2)


def kernel(data, edge_index, W1, b1, W2, b2):
    ei = edge_index.astype(jnp.int32)
    src = ei[0].reshape(E // CHUNK, CHUNK)
    dst = ei[1].reshape(E // CHUNK, CHUNK)

    zeros_deg = jnp.zeros((DPT,), jnp.float32)
    zeros128 = jnp.zeros((SPT, HIDDEN), jnp.float32)
    zeros64 = jnp.zeros((SPT, CLASSES), jnp.float32)

    deg_parts = _deg_call(dst, zeros_deg)
    d0 = deg_parts[0 * N_PAD:0 * N_PAD + N].reshape(N, 1)
    d1 = deg_parts[1 * N_PAD:1 * N_PAD + N].reshape(N, 1)

    xs1, dinv = _tc_layer1(data, W1, d0, d1)

    agg1 = _agg128_call(xs1, src, dst, zeros128)
    xs2 = _tc_layer2(agg1[:N], agg1[N_ACC:N_ACC + N], xs1, dinv,
                     b1.reshape(1, HIDDEN), W2)

    agg2 = _agg64_call(xs2, src, dst, zeros64)
    out = _tc_final(agg2[:N], agg2[N_ACC:N_ACC + N], xs2, dinv,
                    b2.reshape(1, CLASSES))
    return out
